# baseline (speedup 1.0000x reference)
import jax
import jax.numpy as jnp
from jax.experimental import pallas as pl

B, C, H, W = 32, 768, 32, 32
CB = 256


def _bias_add_kernel(x_ref, htT_ref, vtT_ref, reg_ref, out_ref, reg_out_ref):
    htT = htT_ref[...]
    vtT = vtT_ref[...]
    out_ref[0] = x_ref[0] + htT[:, :, None] + vtT[:, None, :]

    @pl.when((pl.program_id(0) == 0) & (pl.program_id(1) == 0))
    def _():
        reg_out_ref[...] = reg_ref[...]


def kernel(x, register_table, vertical_table, horizontal_table):
    Bb, Cc, Hh, Ww = x.shape
    htT = jnp.transpose(horizontal_table[:Hh])
    vtT = jnp.transpose(vertical_table[:Ww])

    grid = (Bb, Cc // CB)
    out, reg_out = pl.pallas_call(
        _bias_add_kernel,
        grid=grid,
        in_specs=[
            pl.BlockSpec((1, CB, Hh, Ww), lambda b, c: (b, c, 0, 0)),
            pl.BlockSpec((CB, Hh), lambda b, c: (c, 0)),
            pl.BlockSpec((CB, Ww), lambda b, c: (c, 0)),
            pl.BlockSpec(register_table.shape, lambda b, c: (0, 0)),
        ],
        out_specs=[
            pl.BlockSpec((1, CB, Hh, Ww), lambda b, c: (b, c, 0, 0)),
            pl.BlockSpec(register_table.shape, lambda b, c: (0, 0)),
        ],
        out_shape=[
            jax.ShapeDtypeStruct(x.shape, x.dtype),
            jax.ShapeDtypeStruct(register_table.shape, register_table.dtype),
        ],
    )(x, htT, vtT, register_table)
    return (out, reg_out)


# speedup vs baseline: 1.5507x; 1.5507x over previous
import jax
import jax.numpy as jnp
from jax.experimental import pallas as pl
from jax.experimental.pallas import tpu as pltpu

B, C, H, W = 32, 768, 32, 32


def _bias_add_kernel(x_ref, htT_ref, vtT_ref, reg_ref, out_ref, reg_out_ref,
                     bias_ref):
    @pl.when(pl.program_id(0) == 0)
    def _():
        htT = htT_ref[...]
        vtT = vtT_ref[...]
        pieces = [htT[:, h:h + 1] + vtT for h in range(H)]
        bias_ref[...] = jnp.concatenate(pieces, axis=1)
        reg_out_ref[...] = reg_ref[...]

    out_ref[...] = x_ref[...] + bias_ref[...]


def kernel(x, register_table, vertical_table, horizontal_table):
    Bb, Cc, Hh, Ww = x.shape
    x2 = x.reshape(Bb * Cc, Hh * Ww)
    htT = jnp.transpose(horizontal_table[:Hh])
    vtT = jnp.transpose(vertical_table[:Ww])

    out2, reg_out = pl.pallas_call(
        _bias_add_kernel,
        grid=(Bb,),
        in_specs=[
            pl.BlockSpec((Cc, Hh * Ww), lambda b: (b, 0)),
            pl.BlockSpec((Cc, Hh), lambda b: (0, 0)),
            pl.BlockSpec((Cc, Ww), lambda b: (0, 0)),
            pl.BlockSpec(register_table.shape, lambda b: (0, 0)),
        ],
        out_specs=[
            pl.BlockSpec((Cc, Hh * Ww), lambda b: (b, 0)),
            pl.BlockSpec(register_table.shape, lambda b: (0, 0)),
        ],
        out_shape=[
            jax.ShapeDtypeStruct(x2.shape, x.dtype),
            jax.ShapeDtypeStruct(register_table.shape, register_table.dtype),
        ],
        scratch_shapes=[pltpu.VMEM((Cc, Hh * Ww), jnp.float32)],
    )(x2, htT, vtT, register_table)
    return (out2.reshape(x.shape), reg_out)


# speedup vs baseline: 3.4605x; 2.2316x over previous
import jax
import jax.numpy as jnp
from jax.experimental import pallas as pl
from jax.experimental.pallas import tpu as pltpu

B, C, H, W = 32, 768, 32, 32


def _bias_add_kernel(x_ref, ht_ref, vt_ref, reg_ref, out_ref, reg_out_ref,
                     bias_ref):
    @pl.when(pl.program_id(0) == 0)
    def _():
        htT = ht_ref[...].T
        vtT = vt_ref[...].T
        for s in range(8):
            pieces = [htT[:, 4 * s + k:4 * s + k + 1] + vtT for k in range(4)]
            bias_ref[:, s, :] = jnp.concatenate(pieces, axis=1)
        reg_out_ref[...] = reg_ref[...]

    out_ref[0] = x_ref[0] + bias_ref[...]


def kernel(x, register_table, vertical_table, horizontal_table):
    Bb, Cc, Hh, Ww = x.shape
    x3 = x.reshape(Bb, Cc, (Hh * Ww) // 128, 128)

    out3, reg_out = pl.pallas_call(
        _bias_add_kernel,
        grid=(Bb,),
        in_specs=[
            pl.BlockSpec((1, Cc, 8, 128), lambda b: (b, 0, 0, 0)),
            pl.BlockSpec(horizontal_table.shape, lambda b: (0, 0)),
            pl.BlockSpec(vertical_table.shape, lambda b: (0, 0)),
            pl.BlockSpec(register_table.shape, lambda b: (0, 0)),
        ],
        out_specs=[
            pl.BlockSpec((1, Cc, 8, 128), lambda b: (b, 0, 0, 0)),
            pl.BlockSpec(register_table.shape, lambda b: (0, 0)),
        ],
        out_shape=[
            jax.ShapeDtypeStruct(x3.shape, x.dtype),
            jax.ShapeDtypeStruct(register_table.shape, register_table.dtype),
        ],
        scratch_shapes=[pltpu.VMEM((Cc, 8, 128), jnp.float32)],
    )(x3, horizontal_table, vertical_table, register_table)
    return (out3.reshape(x.shape), reg_out)


# speedup vs baseline: 3.4948x; 1.0099x over previous
import jax
import jax.numpy as jnp
from jax.experimental import pallas as pl
from jax.experimental.pallas import tpu as pltpu

B, C, H, W = 32, 768, 32, 32


def _bias_add_kernel(x_ref, ht_ref, vt_ref, reg_ref, out_ref, reg_out_ref,
                     bias_ref):
    @pl.when(pl.program_id(0) == 0)
    def _():
        htT = ht_ref[...].T
        vtT = vt_ref[...].T
        for s in range(8):
            pieces = [htT[:, 4 * s + k:4 * s + k + 1] + vtT for k in range(4)]
            bias_ref[:, s, :] = jnp.concatenate(pieces, axis=1)
        reg_out_ref[...] = reg_ref[...]

    out_ref[0] = x_ref[0] + bias_ref[...]
    out_ref[1] = x_ref[1] + bias_ref[...]


def kernel(x, register_table, vertical_table, horizontal_table):
    Bb, Cc, Hh, Ww = x.shape
    x3 = x.reshape(Bb, Cc, (Hh * Ww) // 128, 128)

    out3, reg_out = pl.pallas_call(
        _bias_add_kernel,
        grid=(Bb // 2,),
        in_specs=[
            pl.BlockSpec((2, Cc, 8, 128), lambda b: (b, 0, 0, 0)),
            pl.BlockSpec(horizontal_table.shape, lambda b: (0, 0)),
            pl.BlockSpec(vertical_table.shape, lambda b: (0, 0)),
            pl.BlockSpec(register_table.shape, lambda b: (0, 0)),
        ],
        out_specs=[
            pl.BlockSpec((2, Cc, 8, 128), lambda b: (b, 0, 0, 0)),
            pl.BlockSpec(register_table.shape, lambda b: (0, 0)),
        ],
        out_shape=[
            jax.ShapeDtypeStruct(x3.shape, x.dtype),
            jax.ShapeDtypeStruct(register_table.shape, register_table.dtype),
        ],
        scratch_shapes=[pltpu.VMEM((Cc, 8, 128), jnp.float32)],
    )(x3, horizontal_table, vertical_table, register_table)
    return (out3.reshape(x.shape), reg_out)
